# baseline (speedup 1.0000x reference)
import jax
import jax.numpy as jnp
from jax import lax
from jax.experimental import pallas as pl
from jax.experimental.pallas import tpu as pltpu
from jax.experimental.pallas import tpu_sc as plsc

N = 10000
E = 320000
D = 128
H = 16

NC, NS = 2, 16
NW = NC * NS
EPW = E // NW
CHUNK = 80
NCH = EPW // CHUNK
ECH = E // CHUNK
RB = 5
GRP = NCH // RB
NP = 10240
RPT = NP // NS
PR = N * H // 128

_mesh = plsc.VectorSubcoreMesh(
    core_axis_name="c", subcore_axis_name="s", num_cores=NC, num_subcores=NS)


def _seg_body(y, eic, out0, out1, cnt0, cnt1,
              srcall, dstall, rows, ones, zb, zb1, csv, csbf,
              acc, cacc, gsems, ssems):
    cid = lax.axis_index("c")
    sid = lax.axis_index("s")
    wid = sid * NC + cid

    z16 = jnp.zeros((16,), jnp.float32)
    o16 = jnp.ones((16,), jnp.float32)
    for i in range(zb.shape[0]):
        zb[i] = z16
    for i in range(RPT // 16):
        zb1[pl.ds(i * 16, 16)] = z16
    for i in range(CHUNK // 16):
        ones[pl.ds(i * 16, 16)] = o16

    pltpu.sync_copy(eic.at[pl.ds(wid * NCH, NCH)], srcall)
    pltpu.sync_copy(eic.at[pl.ds(ECH + wid * NCH, NCH)], dstall)

    rbase = sid * RPT
    for j in range(RPT // 64):
        pltpu.sync_copy(zb, acc.at[pl.ds(rbase + j * 64, 64)])
    pltpu.sync_copy(zb1, cacc.at[pl.ds(rbase, RPT)])
    plsc.subcore_barrier()

    for b in range(RB):
        pltpu.async_copy(y.at[srcall.at[b]], rows.at[b], gsems.at[b])

    def body(g, carry):
        for b in range(RB):
            c = g * RB + b
            pltpu.make_async_copy(y.at[srcall.at[c]], rows.at[b],
                                  gsems.at[b]).wait()
            d1 = pltpu.async_copy(rows.at[b], acc.at[dstall.at[c]],
                                  ssems.at[b], add=True)
            d2 = pltpu.async_copy(ones, cacc.at[dstall.at[c]],
                                  ssems.at[b], add=True)
            d1.wait()
            d2.wait()

            @pl.when(c + RB < NCH)
            def _():
                pltpu.async_copy(y.at[srcall.at[c + RB]], rows.at[b],
                                 gsems.at[b])
        return carry

    lax.fori_loop(0, GRP, body, 0)

    plsc.subcore_barrier()

    pltpu.sync_copy(cacc.at[pl.ds(rbase, RPT)], csv)

    def sbody(g, carry):
        c16 = csv[pl.ds(g * 16, 16)]
        for k in range(16):
            spl = jnp.take_along_axis(c16, jnp.full((16,), k, jnp.int32),
                                      axis=0)
            csbf[pl.ds(g * 256 + k * 16, 16)] = spl
        return carry

    lax.fori_loop(0, RPT // 16, sbody, 0)

    last = (NS - 1) * RPT
    lastn = N - last

    @pl.when(jnp.logical_and(sid < NS - 1, cid == 0))
    def _():
        pltpu.sync_copy(acc.at[pl.ds(rbase, RPT)], out0.at[pl.ds(rbase, RPT)])
        pltpu.sync_copy(csbf, cnt0.at[pl.ds(rbase * 16, RPT * 16)])

    @pl.when(jnp.logical_and(sid == NS - 1, cid == 0))
    def _():
        pltpu.sync_copy(acc.at[pl.ds(last, lastn)], out0.at[pl.ds(last, lastn)])
        pltpu.sync_copy(csbf.at[pl.ds(0, lastn * 16)],
                        cnt0.at[pl.ds(last * 16, lastn * 16)])

    @pl.when(jnp.logical_and(sid < NS - 1, cid == 1))
    def _():
        pltpu.sync_copy(acc.at[pl.ds(rbase, RPT)], out1.at[pl.ds(rbase, RPT)])
        pltpu.sync_copy(csbf, cnt1.at[pl.ds(rbase * 16, RPT * 16)])

    @pl.when(jnp.logical_and(sid == NS - 1, cid == 1))
    def _():
        pltpu.sync_copy(acc.at[pl.ds(last, lastn)], out1.at[pl.ds(last, lastn)])
        pltpu.sync_copy(csbf.at[pl.ds(0, lastn * 16)],
                        cnt1.at[pl.ds(last * 16, lastn * 16)])


_seg_call = pl.kernel(
    _seg_body,
    out_type=(
        jax.ShapeDtypeStruct((N, H), jnp.float32),
        jax.ShapeDtypeStruct((N, H), jnp.float32),
        jax.ShapeDtypeStruct((N * H,), jnp.float32),
        jax.ShapeDtypeStruct((N * H,), jnp.float32),
    ),
    mesh=_mesh,
    scratch_types=[
        pltpu.VMEM((NCH, CHUNK), jnp.int32),
        pltpu.VMEM((NCH, CHUNK), jnp.int32),
        pltpu.VMEM((RB, CHUNK, H), jnp.float32),
        pltpu.VMEM((CHUNK,), jnp.float32),
        pltpu.VMEM((64, H), jnp.float32),
        pltpu.VMEM((RPT,), jnp.float32),
        pltpu.VMEM((RPT,), jnp.float32),
        pltpu.VMEM((RPT * 16,), jnp.float32),
        pltpu.VMEM_SHARED((NP, H), jnp.float32),
        pltpu.VMEM_SHARED((NP,), jnp.float32),
        pltpu.SemaphoreType.DMA((RB,)),
        pltpu.SemaphoreType.DMA((RB,)),
    ],
    compiler_params=pltpu.CompilerParams(use_tc_tiling_on_sc=False),
)


def _proj_body(x_ref, wl_ref, wr_ref, y_ref, z_ref):
    xb = x_ref[...]
    y_ref[...] = jnp.dot(xb, wl_ref[...], preferred_element_type=jnp.float32)
    z_ref[...] = jnp.dot(xb, wr_ref[...], preferred_element_type=jnp.float32)


def _comb_body(p0, p1, c0, c1, z, blp, g, b, wl, wr, y_ref, z_ref):
    m = (p0[...] + p1[...]) / jnp.maximum(c0[...] + c1[...], 1.0)
    h = (m + z[...] + blp[...]) * g[...] + b[...]
    h = jnp.maximum(h, 0.0)
    y_ref[...] = jnp.dot(h, wl[...], preferred_element_type=jnp.float32)
    z_ref[...] = jnp.dot(h, wr[...], preferred_element_type=jnp.float32)


def _hfin_body(p0, p1, c0, c1, z, blp, h_ref):
    h_ref[...] = (p0[...] + p1[...]) / jnp.maximum(c0[...] + c1[...], 1.0) \
        + z[...] + blp[...]


def _lsm(t):
    mx = jnp.max(t, axis=1, keepdims=True)
    e = jnp.exp(t - mx)
    return (t - mx) - jnp.log(jnp.sum(e, axis=1, keepdims=True))


def _heads_body(h_ref, w1, b1, w2, b2, o1, o2, o3):
    h = h_ref[...]
    t1 = jnp.dot(h, w1[...], preferred_element_type=jnp.float32) + b1[...]
    o1[...] = _lsm(t1)
    t2 = jnp.dot(h, w2[...], preferred_element_type=jnp.float32) + b2[...]
    o2[...] = _lsm(t2)
    o3[...] = _lsm(h)


def kernel(x, edge_index0, edge_index1, edge_index2, W_l0, b_l0, W_r0,
           W_l1, b_l1, W_r1, W_l2, b_l2, W_r2, gamma0, beta0, gamma1, beta1,
           head1_W, head1_b, head2_W, head2_b):
    f32 = jnp.float32
    eye8 = jnp.eye(8, dtype=f32)
    bn_s = 1.0 / jnp.sqrt(jnp.asarray(1.0 + 1e-5, f32))

    def tile8(v):
        return jnp.tile(v, 8).reshape(1, -1)

    sds = jax.ShapeDtypeStruct

    y0, z0 = pl.pallas_call(
        _proj_body,
        out_shape=[sds((N, H), f32)] * 2,
    )(x, W_l0.T, W_r0.T)
    y0p = y0.reshape(PR, 128)
    z0p = z0.reshape(PR, 128)

    def agg(yp, ei):
        p0, p1, c0f, c1f = _seg_call(yp.reshape(N, H), ei.reshape(2 * ECH, CHUNK))
        return (p0.reshape(PR, 128), p1.reshape(PR, 128),
                c0f.reshape(PR, 128), c1f.reshape(PR, 128))

    def comb(parts, zp, blp, gam, bet, Wl, Wr):
        p0, p1, c0, c1 = parts
        return pl.pallas_call(
            _comb_body,
            out_shape=[sds((PR, 128), f32)] * 2,
        )(p0, p1, c0, c1, zp, tile8(blp), tile8(gam * bn_s), tile8(bet),
          jnp.kron(eye8, Wl.T), jnp.kron(eye8, Wr.T))

    parts0 = agg(y0p, edge_index0)
    y1p, z1p = comb(parts0, z0p, b_l0, gamma0, beta0, W_l1, W_r1)
    parts1 = agg(y1p, edge_index1)
    y2p, z2p = comb(parts1, z1p, b_l1, gamma1, beta1, W_l2, W_r2)
    p0, p1, c0, c1 = agg(y2p, edge_index2)

    hp = pl.pallas_call(
        _hfin_body,
        out_shape=sds((PR, 128), f32),
    )(p0, p1, c0, c1, z2p, tile8(b_l2))

    out1, out2, hls = pl.pallas_call(
        _heads_body,
        out_shape=[sds((N, 4), f32), sds((N, 3), f32), sds((N, H), f32)],
    )(hp.reshape(N, H), head1_W.T, head1_b.reshape(1, 4),
      head2_W.T, head2_b.reshape(1, 3))

    return (out1, out2, hls)


# speedup vs baseline: 1.3103x; 1.3103x over previous
import jax
import jax.numpy as jnp
from jax import lax
from jax.experimental import pallas as pl
from jax.experimental.pallas import tpu as pltpu
from jax.experimental.pallas import tpu_sc as plsc

N = 10000
E = 320000
D = 128
H = 16

NC, NS = 2, 16
NW = NC * NS
EPW = E // NW
CHUNK = 80
NCH = EPW // CHUNK
ECH = E // CHUNK
RB = 10
LA = 8
RB2 = 5
NP = 10240
RPT = NP // NS
PR = N * H // 128

_mesh = plsc.VectorSubcoreMesh(
    core_axis_name="c", subcore_axis_name="s", num_cores=NC, num_subcores=NS)


def _seg_body(y, eic, out0, out1, cnt0, cnt1,
              srcall, dstall, rows, ones, zb, zb1, csv, csbf, dumb,
              acc, cacc, gsems, ssems, csem):
    cid = lax.axis_index("c")
    sid = lax.axis_index("s")
    wid = sid * NC + cid

    z16 = jnp.zeros((16,), jnp.float32)
    o16 = jnp.ones((16,), jnp.float32)
    for i in range(zb.shape[0]):
        zb[i] = z16
    for i in range(RPT // 16):
        zb1[pl.ds(i * 16, 16)] = z16
    for i in range(CHUNK // 16):
        ones[pl.ds(i * 16, 16)] = o16

    pltpu.sync_copy(eic.at[pl.ds(wid * NCH, NCH)], srcall)
    pltpu.sync_copy(eic.at[pl.ds(ECH + wid * NCH, NCH)], dstall)

    rbase = sid * RPT
    for j in range(RPT // 64):
        pltpu.sync_copy(zb, acc.at[pl.ds(rbase + j * 64, 64)])
    pltpu.sync_copy(zb1, cacc.at[pl.ds(rbase, RPT)])
    plsc.subcore_barrier()

    for b in range(LA):
        pltpu.async_copy(y.at[srcall.at[b]], rows.at[b], gsems.at[b])

    def body(g, carry):
        bofs = (g % 2) * RB2
        for p in range(RB2):
            c = g * RB2 + p
            b = bofs + p
            pltpu.make_async_copy(y.at[srcall.at[c]], rows.at[b],
                                  gsems.at[b]).wait()
            pltpu.async_copy(rows.at[b], acc.at[dstall.at[c]],
                             ssems.at[b], add=True)
            pltpu.async_copy(ones, cacc.at[dstall.at[c]], csem, add=True)
            bn = jax.lax.rem(b + LA, RB)

            @pl.when(c >= RB - LA)
            def _():
                pltpu.make_async_copy(rows.at[bn],
                                      acc.at[dstall.at[c - (RB - LA)]],
                                      ssems.at[bn]).wait()

            @pl.when(c + LA < NCH)
            def _():
                pltpu.async_copy(y.at[srcall.at[c + LA]], rows.at[bn],
                                 gsems.at[bn])
        return carry

    lax.fori_loop(0, NCH // RB2, body, 0)

    for c in range(NCH - (RB - LA), NCH):
        pltpu.make_async_copy(rows.at[c % RB], acc.at[dstall.at[c]],
                              ssems.at[c % RB]).wait()
    pltpu.make_async_copy(y.at[pl.ds(0, NCH * CHUNK // H)], dumb, csem).wait()

    plsc.subcore_barrier()

    pltpu.sync_copy(cacc.at[pl.ds(rbase, RPT)], csv)

    def sbody(g, carry):
        c16 = csv[pl.ds(g * 16, 16)]
        for k in range(16):
            spl = jnp.take_along_axis(c16, jnp.full((16,), k, jnp.int32),
                                      axis=0)
            csbf[pl.ds(g * 256 + k * 16, 16)] = spl
        return carry

    lax.fori_loop(0, RPT // 16, sbody, 0)

    last = (NS - 1) * RPT
    lastn = N - last

    @pl.when(jnp.logical_and(sid < NS - 1, cid == 0))
    def _():
        pltpu.sync_copy(acc.at[pl.ds(rbase, RPT)], out0.at[pl.ds(rbase, RPT)])
        pltpu.sync_copy(csbf, cnt0.at[pl.ds(rbase * 16, RPT * 16)])

    @pl.when(jnp.logical_and(sid == NS - 1, cid == 0))
    def _():
        pltpu.sync_copy(acc.at[pl.ds(last, lastn)], out0.at[pl.ds(last, lastn)])
        pltpu.sync_copy(csbf.at[pl.ds(0, lastn * 16)],
                        cnt0.at[pl.ds(last * 16, lastn * 16)])

    @pl.when(jnp.logical_and(sid < NS - 1, cid == 1))
    def _():
        pltpu.sync_copy(acc.at[pl.ds(rbase, RPT)], out1.at[pl.ds(rbase, RPT)])
        pltpu.sync_copy(csbf, cnt1.at[pl.ds(rbase * 16, RPT * 16)])

    @pl.when(jnp.logical_and(sid == NS - 1, cid == 1))
    def _():
        pltpu.sync_copy(acc.at[pl.ds(last, lastn)], out1.at[pl.ds(last, lastn)])
        pltpu.sync_copy(csbf.at[pl.ds(0, lastn * 16)],
                        cnt1.at[pl.ds(last * 16, lastn * 16)])


_seg_call = pl.kernel(
    _seg_body,
    out_type=(
        jax.ShapeDtypeStruct((N, H), jnp.float32),
        jax.ShapeDtypeStruct((N, H), jnp.float32),
        jax.ShapeDtypeStruct((N * H,), jnp.float32),
        jax.ShapeDtypeStruct((N * H,), jnp.float32),
    ),
    mesh=_mesh,
    scratch_types=[
        pltpu.VMEM((NCH, CHUNK), jnp.int32),
        pltpu.VMEM((NCH, CHUNK), jnp.int32),
        pltpu.VMEM((RB, CHUNK, H), jnp.float32),
        pltpu.VMEM((CHUNK,), jnp.float32),
        pltpu.VMEM((64, H), jnp.float32),
        pltpu.VMEM((RPT,), jnp.float32),
        pltpu.VMEM((RPT,), jnp.float32),
        pltpu.VMEM((RPT * 16,), jnp.float32),
        pltpu.VMEM((NCH * CHUNK // H, H), jnp.float32),
        pltpu.VMEM_SHARED((NP, H), jnp.float32),
        pltpu.VMEM_SHARED((NP,), jnp.float32),
        pltpu.SemaphoreType.DMA((RB,)),
        pltpu.SemaphoreType.DMA((RB,)),
        pltpu.SemaphoreType.DMA,
    ],
    compiler_params=pltpu.CompilerParams(use_tc_tiling_on_sc=False),
)


def _proj_body(x_ref, wl_ref, wr_ref, y_ref, z_ref):
    xb = x_ref[...]
    y_ref[...] = jnp.dot(xb, wl_ref[...], preferred_element_type=jnp.float32)
    z_ref[...] = jnp.dot(xb, wr_ref[...], preferred_element_type=jnp.float32)


def _comb_body(p0, p1, c0, c1, z, blp, g, b, wl, wr, y_ref, z_ref):
    m = (p0[...] + p1[...]) / jnp.maximum(c0[...] + c1[...], 1.0)
    h = (m + z[...] + blp[...]) * g[...] + b[...]
    h = jnp.maximum(h, 0.0)
    y_ref[...] = jnp.dot(h, wl[...], preferred_element_type=jnp.float32)
    z_ref[...] = jnp.dot(h, wr[...], preferred_element_type=jnp.float32)


def _lsm_packed(t, gmat):
    mx = jnp.max(t, axis=1, keepdims=True)
    e = jnp.exp(t - mx)
    s = jnp.dot(e, gmat, preferred_element_type=jnp.float32)
    return (t - mx) - jnp.log(s)


def _fin_body(p0, p1, c0, c1, z, blp, w1, b1, w2, b2, g4, g3, g16,
              o1, o2, o3):
    h = (p0[...] + p1[...]) / jnp.maximum(c0[...] + c1[...], 1.0) \
        + z[...] + blp[...]
    t1 = jnp.dot(h, w1[...], preferred_element_type=jnp.float32) + b1[...]
    o1[...] = _lsm_packed(t1, g4[...])
    t2 = jnp.dot(h, w2[...], preferred_element_type=jnp.float32) + b2[...]
    o2[...] = _lsm_packed(t2, g3[...])
    o3[...] = _lsm_packed(h, g16[...])


def kernel(x, edge_index0, edge_index1, edge_index2, W_l0, b_l0, W_r0,
           W_l1, b_l1, W_r1, W_l2, b_l2, W_r2, gamma0, beta0, gamma1, beta1,
           head1_W, head1_b, head2_W, head2_b):
    f32 = jnp.float32
    eye8 = jnp.eye(8, dtype=f32)
    bn_s = 1.0 / jnp.sqrt(jnp.asarray(1.0 + 1e-5, f32))

    def tile8(v):
        return jnp.tile(v, 8).reshape(1, -1)

    sds = jax.ShapeDtypeStruct

    y0, z0 = pl.pallas_call(
        _proj_body,
        out_shape=[sds((N, H), f32)] * 2,
    )(x, W_l0.T, W_r0.T)
    y0p = y0.reshape(PR, 128)
    z0p = z0.reshape(PR, 128)

    def agg(yp, ei):
        p0, p1, c0f, c1f = _seg_call(yp.reshape(N, H), ei.reshape(2 * ECH, CHUNK))
        return (p0.reshape(PR, 128), p1.reshape(PR, 128),
                c0f.reshape(PR, 128), c1f.reshape(PR, 128))

    def comb(parts, zp, blp, gam, bet, Wl, Wr):
        p0, p1, c0, c1 = parts
        return pl.pallas_call(
            _comb_body,
            out_shape=[sds((PR, 128), f32)] * 2,
        )(p0, p1, c0, c1, zp, tile8(blp), tile8(gam * bn_s), tile8(bet),
          jnp.kron(eye8, Wl.T), jnp.kron(eye8, Wr.T))

    parts0 = agg(y0p, edge_index0)
    y1p, z1p = comb(parts0, z0p, b_l0, gamma0, beta0, W_l1, W_r1)
    parts1 = agg(y1p, edge_index1)
    y2p, z2p = comb(parts1, z1p, b_l1, gamma1, beta1, W_l2, W_r2)
    p0, p1, c0, c1 = agg(y2p, edge_index2)

    o1p, o2p, hlsp = pl.pallas_call(
        _fin_body,
        out_shape=[sds((PR, 32), f32), sds((PR, 24), f32),
                   sds((PR, 128), f32)],
    )(p0, p1, c0, c1, z2p, tile8(b_l2),
      jnp.kron(eye8, head1_W.T), tile8(head1_b),
      jnp.kron(eye8, head2_W.T), tile8(head2_b),
      jnp.kron(eye8, jnp.ones((4, 4), f32)),
      jnp.kron(eye8, jnp.ones((3, 3), f32)),
      jnp.kron(eye8, jnp.ones((16, 16), f32)))

    return (o1p.reshape(N, 4), o2p.reshape(N, 3), hlsp.reshape(N, H))
